# baseline (speedup 1.0000x reference)
import functools

import jax
import jax.numpy as jnp
from jax import lax
from jax.experimental import pallas as pl
from jax.experimental.pallas import tpu as pltpu
from jax.experimental.pallas import tpu_sc as plsc

B = 16384
E = 64
V = 100000
L = 16
NC = 2
NS = 16
NW = NC * NS
S0 = 50048
S1 = 49920
VMAIN = S0 + S1
VTAIL = V - VMAIN
HB = S0
IQ = B // 4
T_U, T_M, T_UB, T_MB, T_LEN = 0, 2048, 4096, 4128, 5120


def _sc_main(idx_t, uembt, membt, ubt, mbt, tails):
    @functools.partial(
        pl.kernel,
        out_type=[
            jax.ShapeDtypeStruct((NW, 128), jnp.float32),
            jax.ShapeDtypeStruct((B,), jnp.float32),
            jax.ShapeDtypeStruct((B,), jnp.float32),
        ],
        mesh=plsc.VectorSubcoreMesh(core_axis_name="c", subcore_axis_name="s"),
        compiler_params=pltpu.CompilerParams(needs_layout_passes=False),
        scratch_types=[
            pltpu.VMEM((HB,), jnp.float32),
            pltpu.VMEM((HB,), jnp.float32),
            pltpu.VMEM((B,), jnp.float32),
            pltpu.VMEM((IQ,), jnp.int32),
            pltpu.VMEM((IQ,), jnp.int32),
            pltpu.VMEM((T_LEN,), jnp.float32),
            pltpu.SemaphoreType.DMA,
            pltpu.SemaphoreType.DMA,
            pltpu.SemaphoreType.DMA,
        ],
    )
    def k(idx_hbm, uembt_hbm, membt_hbm, ubt_hbm, mbt_hbm, tails_hbm,
          partials_out, bu_out, bm_out,
          rowA, rowB, uvec, idxa, idxb, tails_v, semA, semB, semi):
        wid = lax.axis_index("s") * NC + lax.axis_index("c")
        lanes = jnp.arange(L, dtype=jnp.int32)
        pltpu.sync_copy(tails_hbm, tails_v)
        ibufs = (idxa, idxb)

        def stage_A(src2d, row):
            return pltpu.async_copy(src2d.at[row, pl.ds(0, S0)],
                                    rowA.at[pl.ds(0, S0)], semA)

        def stage_B(src2d, row):
            return pltpu.async_copy(src2d.at[row, pl.ds(S0, S1)],
                                    rowB.at[pl.ds(0, S1)], semB)

        def inject_tail(row, tbase, is_bias):
            for c in range(VTAIL // L):
                rel = lanes + c * L
                fidx = tbase + (rel if is_bias else rel * E + row)
                rowB[pl.ds(S1 + c * L, L)] = plsc.load_gather(tails_v, [fidx])

        def half_pass(idx_row, half, kind, acc0):
            rbuf = rowA if half == 0 else rowB
            zero = jnp.zeros((L,), jnp.float32)

            def chunk(off, acc):
                ic = ibufs_cur[0][pl.ds(off, L)]
                if half == 0:
                    mask = ic < S0
                    cl = jnp.minimum(ic, S0 - 1)
                else:
                    rel = ic - S0
                    mask = rel >= 0
                    cl = jnp.maximum(rel, 0)
                g = jnp.where(mask, plsc.load_gather(rbuf, [cl]), zero)
                if kind == 0:
                    uvec[pl.ds(uq_base + off, L)] = g
                elif kind == 1:
                    plsc.addupdate(uvec.at[pl.ds(uq_base + off, L)], g)
                else:
                    acc = acc + g * uvec[pl.ds(uq_base + off, L)]
                return acc

            acc = acc0
            cp = pltpu.async_copy(idx_hbm.at[idx_row, pl.ds(0, IQ)], ibufs[0],
                                  semi)
            for q in range(4):
                cp.wait()
                if q < 3:
                    cp = pltpu.async_copy(
                        idx_hbm.at[idx_row, pl.ds((q + 1) * IQ, IQ)],
                        ibufs[(q + 1) % 2], semi)
                ibufs_cur = (ibufs[q % 2],)
                uq_base = q * IQ

                def body(i, a):
                    for t in range(8):
                        a = chunk(i * 128 + t * L, a)
                    return a
                acc = lax.fori_loop(0, IQ // 128, body, acc)
            return acc

        def bias_job(bt_hbm, idx_row, tbase, q, out_hbm):
            cpA = stage_A(bt_hbm, 0)
            cpB = stage_B(bt_hbm, 0)
            cpA.wait()
            cpB.wait()
            inject_tail(0, tbase, True)
            pltpu.sync_copy(idx_hbm.at[idx_row, pl.ds(q * IQ, IQ)], idxa)
            zero = jnp.zeros((L,), jnp.float32)

            def body(i, _):
                for t in range(8):
                    off = i * 128 + t * L
                    ic = idxa[pl.ds(off, L)]
                    mask = ic < S0
                    ga = plsc.load_gather(rowA, [jnp.minimum(ic, S0 - 1)])
                    gb = plsc.load_gather(rowB, [jnp.maximum(ic - S0, 0)])
                    uvec[pl.ds(off, L)] = jnp.where(mask, ga, gb)
                return 0
            lax.fori_loop(0, IQ // 128, body, 0)
            pltpu.sync_copy(uvec.at[pl.ds(0, IQ)], out_hbm.at[pl.ds(q * IQ, IQ)])

        @pl.when(wid < 4)
        def _():
            bias_job(ubt_hbm, 0, T_UB, wid, bu_out)

        @pl.when((wid >= 4) & (wid < 8))
        def _():
            bias_job(mbt_hbm, 1, T_MB, wid - 4, bm_out)

        acc = jnp.zeros((L,), jnp.float32)
        cpA = stage_A(uembt_hbm, wid * 2)
        for j in range(2):
            e = wid * 2 + j
            cpB = stage_B(uembt_hbm, e)
            cpA.wait()
            half_pass(0, 0, 0, None)
            cpB.wait()
            inject_tail(e, T_U, False)
            cpA = stage_A(membt_hbm, e)
            half_pass(0, 1, 1, None)
            cpA.wait()
            cpB = stage_B(membt_hbm, e)
            acc = half_pass(1, 0, 2, acc)
            cpB.wait()
            inject_tail(e, T_M, False)
            if j == 0:
                cpA = stage_A(uembt_hbm, e + 1)
            acc = half_pass(1, 1, 2, acc)

        o = uvec
        o[pl.ds(0, L)] = acc
        zero = jnp.zeros((L,), jnp.float32)
        for c in range(1, 128 // L):
            o[pl.ds(c * L, L)] = zero
        pltpu.sync_copy(o.at[pl.ds(0, 128)], partials_out.at[wid])

    return k(idx_t, uembt, membt, ubt, mbt, tails)


def _tc_finish(partials, bu2d, bm2d):
    def body(p_ref, bu_ref, bm_ref, o_ref):
        s = jnp.sum(p_ref[...])
        o_ref[...] = jax.nn.sigmoid(bu_ref[...] + bm_ref[...] + s)

    return pl.pallas_call(
        body,
        out_shape=jax.ShapeDtypeStruct(bu2d.shape, jnp.float32),
    )(partials, bu2d, bm2d)


def kernel(inputs, user_emb, user_bias_tab, movie_emb, movie_bias_tab):
    tails = jnp.concatenate([
        user_emb[VMAIN:].reshape(-1),
        movie_emb[VMAIN:].reshape(-1),
        user_bias_tab[VMAIN:, 0],
        movie_bias_tab[VMAIN:, 0],
        jnp.zeros((T_LEN - T_MB - VTAIL,), jnp.float32),
    ])
    partials, bu, bm = _sc_main(inputs.T, user_emb.T, movie_emb.T,
                                user_bias_tab.T, movie_bias_tab.T, tails)
    out2d = _tc_finish(partials, bu.reshape(128, 128), bm.reshape(128, 128))
    return out2d.reshape(B, 1)


# speedup vs baseline: 1.2762x; 1.2762x over previous
import functools

import jax
import jax.numpy as jnp
from jax import lax
from jax.experimental import pallas as pl
from jax.experimental.pallas import tpu as pltpu
from jax.experimental.pallas import tpu_sc as plsc

B = 16384
E = 64
V = 100000
L = 16
NC = 2
NS = 16
NW = NC * NS
VMAIN = 99968
VTAIL = V - VMAIN
VPAD = VMAIN + 128
S0 = 50048
S1 = VMAIN - S0
IQ = B // 4
T_U, T_M, T_UB, T_MB, T_LEN = 0, 2048, 4096, 4128, 5120


def _sc_main(idx_t, uembt, membt, ubt, mbt, tails):
    @functools.partial(
        pl.kernel,
        out_type=[
            jax.ShapeDtypeStruct((NW, 128), jnp.float32),
            jax.ShapeDtypeStruct((B,), jnp.float32),
            jax.ShapeDtypeStruct((B,), jnp.float32),
        ],
        mesh=plsc.VectorSubcoreMesh(core_axis_name="c", subcore_axis_name="s"),
        compiler_params=pltpu.CompilerParams(needs_layout_passes=False),
        scratch_types=[
            pltpu.VMEM((VPAD,), jnp.float32),
            pltpu.VMEM((B,), jnp.float32),
            pltpu.VMEM((IQ,), jnp.int32),
            pltpu.VMEM((IQ,), jnp.int32),
            pltpu.VMEM((T_LEN,), jnp.float32),
            pltpu.SemaphoreType.DMA,
            pltpu.SemaphoreType.DMA,
            pltpu.SemaphoreType.DMA,
        ],
    )
    def k(idx_hbm, uembt_hbm, membt_hbm, ubt_hbm, mbt_hbm, tails_hbm,
          partials_out, bu_out, bm_out,
          rowv, uvec, idxa, idxb, tails_v, sem0, sem1, semi):
        wid = lax.axis_index("s") * NC + lax.axis_index("c")
        lanes = jnp.arange(L, dtype=jnp.int32)
        pltpu.sync_copy(tails_hbm, tails_v)
        ibufs = (idxa, idxb)

        def stage_row(src2d, row, tbase, is_bias):
            cp0 = pltpu.async_copy(src2d.at[row, pl.ds(0, S0)],
                                   rowv.at[pl.ds(0, S0)], sem0)
            cp1 = pltpu.async_copy(src2d.at[row, pl.ds(S0, S1)],
                                   rowv.at[pl.ds(S0, S1)], sem1)
            cp0.wait()
            cp1.wait()
            for c in range(VTAIL // L):
                rel = lanes + c * L
                fidx = tbase + (rel if is_bias else rel * E + row)
                rowv[pl.ds(VMAIN + c * L, L)] = plsc.load_gather(tails_v, [fidx])

        def quarter_loop(idx_row, mode, acc0):
            acc = acc0
            cp = pltpu.async_copy(idx_hbm.at[idx_row, pl.ds(0, IQ)], ibufs[0],
                                  semi)
            for q in range(4):
                cp.wait()
                if q < 3:
                    cp = pltpu.async_copy(
                        idx_hbm.at[idx_row, pl.ds((q + 1) * IQ, IQ)],
                        ibufs[(q + 1) % 2], semi)
                idxv = ibufs[q % 2]
                base = q * IQ

                def body(i, a):
                    for t in range(16):
                        off = i * 256 + t * L
                        ic = idxv[pl.ds(off, L)]
                        g = plsc.load_gather(rowv, [ic])
                        if mode == 0:
                            uvec[pl.ds(base + off, L)] = g
                        else:
                            a = a + g * uvec[pl.ds(base + off, L)]
                    return a
                acc = lax.fori_loop(0, IQ // 256, body, acc)
            return acc

        def bias_job(bt_hbm, idx_row, tbase, q, out_hbm):
            stage_row(bt_hbm, 0, tbase, True)
            pltpu.sync_copy(idx_hbm.at[idx_row, pl.ds(q * IQ, IQ)], idxa)

            def body(i, _):
                for t in range(16):
                    off = i * 256 + t * L
                    ic = idxa[pl.ds(off, L)]
                    uvec[pl.ds(off, L)] = plsc.load_gather(rowv, [ic])
                return 0
            lax.fori_loop(0, IQ // 256, body, 0)
            pltpu.sync_copy(uvec.at[pl.ds(0, IQ)], out_hbm.at[pl.ds(q * IQ, IQ)])

        @pl.when(wid < 4)
        def _():
            bias_job(ubt_hbm, 0, T_UB, wid, bu_out)

        @pl.when((wid >= 4) & (wid < 8))
        def _():
            bias_job(mbt_hbm, 1, T_MB, wid - 4, bm_out)

        acc = jnp.zeros((L,), jnp.float32)
        for j in range(2):
            e = wid * 2 + j
            stage_row(uembt_hbm, e, T_U, False)
            quarter_loop(0, 0, None)
            stage_row(membt_hbm, e, T_M, False)
            acc = quarter_loop(1, 1, acc)

        o = uvec
        o[pl.ds(0, L)] = acc
        zero = jnp.zeros((L,), jnp.float32)
        for c in range(1, 128 // L):
            o[pl.ds(c * L, L)] = zero
        pltpu.sync_copy(o.at[pl.ds(0, 128)], partials_out.at[wid])

    return k(idx_t, uembt, membt, ubt, mbt, tails)


def _tc_finish(partials, bu2d, bm2d):
    def body(p_ref, bu_ref, bm_ref, o_ref):
        s = jnp.sum(p_ref[...])
        o_ref[...] = jax.nn.sigmoid(bu_ref[...] + bm_ref[...] + s)

    return pl.pallas_call(
        body,
        out_shape=jax.ShapeDtypeStruct(bu2d.shape, jnp.float32),
    )(partials, bu2d, bm2d)


def kernel(inputs, user_emb, user_bias_tab, movie_emb, movie_bias_tab):
    tails = jnp.concatenate([
        user_emb[VMAIN:].reshape(-1),
        movie_emb[VMAIN:].reshape(-1),
        user_bias_tab[VMAIN:, 0],
        movie_bias_tab[VMAIN:, 0],
        jnp.zeros((T_LEN - T_MB - VTAIL,), jnp.float32),
    ])
    partials, bu, bm = _sc_main(inputs.T, user_emb.T, movie_emb.T,
                                user_bias_tab.T, movie_bias_tab.T, tails)
    out2d = _tc_finish(partials, bu.reshape(128, 128), bm.reshape(128, 128))
    return out2d.reshape(B, 1)


# speedup vs baseline: 1.3666x; 1.0708x over previous
import functools

import jax
import jax.numpy as jnp
from jax import lax
from jax.experimental import pallas as pl
from jax.experimental.pallas import tpu as pltpu
from jax.experimental.pallas import tpu_sc as plsc

B = 16384
E = 64
V = 100000
L = 16
NC = 2
NS = 16
NW = NC * NS
VMAIN = 99968
VTAIL = V - VMAIN
VPAD = VMAIN + 128
S0 = 50048
S1 = VMAIN - S0
IQ = B // 4
T_U, T_M, T_UB, T_MB, T_LEN = 0, 2048, 4096, 4128, 5120


def _sc_main(idx_t, uembt, membt, ubt, mbt, tails):
    @functools.partial(
        pl.kernel,
        out_type=[
            jax.ShapeDtypeStruct((NW, 128), jnp.float32),
            jax.ShapeDtypeStruct((B,), jnp.float32),
            jax.ShapeDtypeStruct((B,), jnp.float32),
        ],
        mesh=plsc.VectorSubcoreMesh(core_axis_name="c", subcore_axis_name="s"),
        compiler_params=pltpu.CompilerParams(needs_layout_passes=False),
        scratch_types=[
            pltpu.VMEM((VPAD,), jnp.float32),
            pltpu.VMEM((B,), jnp.float32),
            pltpu.VMEM((IQ,), jnp.int32),
            pltpu.VMEM((IQ,), jnp.int32),
            pltpu.VMEM((T_LEN,), jnp.float32),
            pltpu.SemaphoreType.DMA,
            pltpu.SemaphoreType.DMA,
            pltpu.SemaphoreType.DMA,
        ],
    )
    def k(idx_hbm, uembt_hbm, membt_hbm, ubt_hbm, mbt_hbm, tails_hbm,
          partials_out, bu_out, bm_out,
          rowv, uvec, idxa, idxb, tails_v, sem0, sem1, semi):
        wid = lax.axis_index("s") * NC + lax.axis_index("c")
        lanes = jnp.arange(L, dtype=jnp.int32)
        pltpu.sync_copy(tails_hbm, tails_v)
        ibufs = (idxa, idxb)

        def stage_row(src2d, row, tbase, is_bias):
            cp0 = pltpu.async_copy(src2d.at[row, pl.ds(0, S0)],
                                   rowv.at[pl.ds(0, S0)], sem0)
            cp1 = pltpu.async_copy(src2d.at[row, pl.ds(S0, S1)],
                                   rowv.at[pl.ds(S0, S1)], sem1)
            cp0.wait()
            cp1.wait()
            for c in range(VTAIL // L):
                rel = lanes + c * L
                fidx = tbase + (rel if is_bias else rel * E + row)
                rowv[pl.ds(VMAIN + c * L, L)] = plsc.load_gather(tails_v, [fidx])

        def quarter_loop(idx_row, mode, acc0):
            acc = acc0
            cp = pltpu.async_copy(idx_hbm.at[idx_row, pl.ds(0, IQ)], ibufs[0],
                                  semi)
            for q in range(4):
                cp.wait()
                if q < 3:
                    cp = pltpu.async_copy(
                        idx_hbm.at[idx_row, pl.ds((q + 1) * IQ, IQ)],
                        ibufs[(q + 1) % 2], semi)
                idxv = ibufs[q % 2]
                base = q * IQ

                if mode == 0:
                    @plsc.parallel_loop(0, IQ, step=L, unroll=16)
                    def _(off):
                        ic = idxv[pl.ds(off, L)]
                        uvec[pl.ds(base + off, L)] = plsc.load_gather(rowv, [ic])
                else:
                    @plsc.parallel_loop(0, IQ, step=2 * L, unroll=8, carry=acc)
                    def acc(off, a):
                        a0, a1 = a
                        ic0 = idxv[pl.ds(off, L)]
                        g0 = plsc.load_gather(rowv, [ic0])
                        a0 = a0 + g0 * uvec[pl.ds(base + off, L)]
                        ic1 = idxv[pl.ds(off + L, L)]
                        g1 = plsc.load_gather(rowv, [ic1])
                        a1 = a1 + g1 * uvec[pl.ds(base + off + L, L)]
                        return (a0, a1)
            return acc

        def bias_job(bt_hbm, idx_row, tbase, q, out_hbm):
            stage_row(bt_hbm, 0, tbase, True)
            pltpu.sync_copy(idx_hbm.at[idx_row, pl.ds(q * IQ, IQ)], idxa)

            @plsc.parallel_loop(0, IQ, step=L, unroll=16)
            def _(off):
                ic = idxa[pl.ds(off, L)]
                uvec[pl.ds(off, L)] = plsc.load_gather(rowv, [ic])
            pltpu.sync_copy(uvec.at[pl.ds(0, IQ)], out_hbm.at[pl.ds(q * IQ, IQ)])

        @pl.when(wid < 4)
        def _():
            bias_job(ubt_hbm, 0, T_UB, wid, bu_out)

        @pl.when((wid >= 4) & (wid < 8))
        def _():
            bias_job(mbt_hbm, 1, T_MB, wid - 4, bm_out)

        acc = (jnp.zeros((L,), jnp.float32), jnp.zeros((L,), jnp.float32))
        for j in range(2):
            e = wid * 2 + j
            stage_row(uembt_hbm, e, T_U, False)
            quarter_loop(0, 0, None)
            stage_row(membt_hbm, e, T_M, False)
            acc = quarter_loop(1, 1, acc)

        o = uvec
        o[pl.ds(0, L)] = acc[0] + acc[1]
        zero = jnp.zeros((L,), jnp.float32)
        for c in range(1, 128 // L):
            o[pl.ds(c * L, L)] = zero
        pltpu.sync_copy(o.at[pl.ds(0, 128)], partials_out.at[wid])

    return k(idx_t, uembt, membt, ubt, mbt, tails)


def _tc_finish(partials, bu2d, bm2d):
    def body(p_ref, bu_ref, bm_ref, o_ref):
        s = jnp.sum(p_ref[...])
        o_ref[...] = jax.nn.sigmoid(bu_ref[...] + bm_ref[...] + s)

    return pl.pallas_call(
        body,
        out_shape=jax.ShapeDtypeStruct(bu2d.shape, jnp.float32),
    )(partials, bu2d, bm2d)


def kernel(inputs, user_emb, user_bias_tab, movie_emb, movie_bias_tab):
    tails = jnp.concatenate([
        user_emb[VMAIN:].reshape(-1),
        movie_emb[VMAIN:].reshape(-1),
        user_bias_tab[VMAIN:, 0],
        movie_bias_tab[VMAIN:, 0],
        jnp.zeros((T_LEN - T_MB - VTAIL,), jnp.float32),
    ])
    partials, bu, bm = _sc_main(inputs.T, user_emb.T, movie_emb.T,
                                user_bias_tab.T, movie_bias_tab.T, tails)
    out2d = _tc_finish(partials, bu.reshape(128, 128), bm.reshape(128, 128))
    return out2d.reshape(B, 1)
